# baseline (speedup 1.0000x reference)
import functools

import jax
import jax.numpy as jnp
from jax import lax
from jax.experimental import pallas as pl
from jax.experimental.pallas import tpu as pltpu
from jax.experimental.pallas import tpu_sc as plsc

EMB_DIM = 32
LANE = 16
BBLK = 128
KCB = 4


@functools.lru_cache(maxsize=None)
def _make_relayout(n_items: int):
    info = plsc.get_sparse_core_info()
    nc, ns = info.num_cores, info.num_subcores
    nw = nc * ns
    ncb = (n_items + 127) // 128
    n_full = (ncb // (KCB * nw)) * KCB
    n_rest = ncb - n_full * nw

    mesh = plsc.VectorSubcoreMesh(core_axis_name="c", subcore_axis_name="s")

    @functools.partial(
        pl.kernel,
        mesh=mesh,
        out_type=jax.ShapeDtypeStruct((ncb * 32, 128), jnp.float32),
        compiler_params=pltpu.CompilerParams(
            use_tc_tiling_on_sc=True, needs_layout_passes=False),
        scratch_types=[
            pltpu.VMEM((4, 8, KCB * 128), jnp.float32),
            pltpu.VMEM((KCB * 32, 128), jnp.float32),
            pltpu.SemaphoreType.DMA,
        ],
    )
    def relayout_kernel(tt_hbm, t2d_hbm, inb, outb, sem):
        w = lax.axis_index("s") * nc + lax.axis_index("c")
        lane = lax.iota(jnp.int32, LANE)
        ebv = [(lane + e0) >> 3 for e0 in (0, 16)]
        erv = [(lane + e0) & 7 for e0 in (0, 16)]

        def do_group(cb0, kcb, width):
            copies = [
                pltpu.async_copy(
                    tt_hbm.at[pl.ds(eb * 8, 8), pl.ds(cb0 * 128, width)],
                    inb.at[eb, :, pl.ds(0, width)], sem)
                for eb in range(4)
            ]
            for cp in copies:
                cp.wait()

            def rbody(r, carry):
                for kk in range(kcb):
                    for m in range(8):
                        half = m % 2
                        s = m // 2
                        col = jnp.full((LANE,), kk * 128 + r * 4 + s,
                                       jnp.int32)
                        v = plsc.load_gather(inb, [ebv[half], erv[half], col])
                        outb[kk * 32 + r, pl.ds(16 * m, LANE)] = v
                return carry

            lax.fori_loop(0, 32, rbody, 0)
            pltpu.sync_copy(
                outb.at[pl.ds(0, kcb * 32)],
                t2d_hbm.at[pl.ds(cb0 * 32, kcb * 32)])

        def gbody(j, carry):
            do_group((w * (n_full // KCB) + j) * KCB, KCB, KCB * 128)
            return carry

        lax.fori_loop(0, n_full // KCB, gbody, 0)
        if n_rest:
            last_w = (n_items - (ncb - 1) * 128) if n_items % 128 else 128

            @pl.when(w < n_rest - 1)
            def _():
                do_group(n_full * nw + w, 1, 128)

            @pl.when(w == n_rest - 1)
            def _():
                do_group(n_full * nw + w, 1, last_w)

        return

    return relayout_kernel


@functools.lru_cache(maxsize=None)
def _make_gather(batch: int, hist: int, n_rows: int):
    info = plsc.get_sparse_core_info()
    nc, ns = info.num_cores, info.num_subcores
    nw = nc * ns
    nbb = batch // BBLK
    assert nbb == nw and batch % BBLK == 0
    neb = EMB_DIM // 8

    mesh = plsc.VectorSubcoreMesh(core_axis_name="c", subcore_axis_name="s")

    @functools.partial(
        pl.kernel,
        mesh=mesh,
        out_type=jax.ShapeDtypeStruct((hist * neb * nbb * 8, BBLK), jnp.float32),
        compiler_params=pltpu.CompilerParams(
            use_tc_tiling_on_sc=False, needs_layout_passes=False),
        scratch_types=[
            pltpu.VMEM((hist, BBLK), jnp.int32),
            pltpu.VMEM((BBLK, EMB_DIM), jnp.float32),
            pltpu.VMEM((neb, 8, BBLK), jnp.float32),
            pltpu.SemaphoreType.DMA,
        ],
    )
    def gather_kernel(idx_hbm, table_hbm, out_hbm, idx_v, rows_v, tile_v, sem):
        w = lax.axis_index("s") * nc + lax.axis_index("c")
        pltpu.sync_copy(idx_hbm.at[:, pl.ds(w * BBLK, BBLK)], idx_v)
        lane = lax.iota(jnp.int32, LANE)

        def body(h, carry):
            pltpu.async_copy(table_hbm.at[idx_v.at[h]], rows_v, sem).wait()
            for e in range(EMB_DIM):
                col = jnp.full((LANE,), e, jnp.int32)
                for i0 in range(BBLK // LANE):
                    v = plsc.load_gather(rows_v, [lane + i0 * LANE, col])
                    tile_v[e // 8, e % 8, pl.ds(i0 * LANE, LANE)] = v
            row0 = ((h * neb) * nbb + w) * 8
            for eb in range(neb):
                pltpu.sync_copy(
                    tile_v.at[eb],
                    out_hbm.at[pl.ds(row0 + eb * nbb * 8, 8)])
            return carry

        lax.fori_loop(0, hist, body, 0)

    return gather_kernel


def kernel(indices, table):
    b, h = indices.shape
    n = table.shape[0]
    idx_t = indices.T.astype(jnp.int32)
    t2d = _make_relayout(n)(table.T)
    n_pad = t2d.shape[0] * 4
    tlin = t2d.reshape(n_pad, EMB_DIM)
    out2d = _make_gather(b, h, n_pad)(idx_t, tlin)
    out5 = out2d.reshape(h, EMB_DIM // 8, b // BBLK, 8, BBLK)
    return out5.transpose(2, 4, 0, 1, 3).reshape(b, h, EMB_DIM)


# speedup vs baseline: 1.0941x; 1.0941x over previous
import functools

import jax
import jax.numpy as jnp
from jax import lax
from jax.experimental import pallas as pl
from jax.experimental.pallas import tpu as pltpu
from jax.experimental.pallas import tpu_sc as plsc

EMB_DIM = 32
LANE = 16
BBLK = 128
KCB = 1
RING = 2


@functools.lru_cache(maxsize=None)
def _make_relayout(n_items: int):
    info = plsc.get_sparse_core_info()
    nc, ns = info.num_cores, info.num_subcores
    nw = nc * ns
    ncb = (n_items + 127) // 128
    ng = ncb // (KCB * nw)
    n_rest = ncb - ng * KCB * nw
    last_w = n_items - (ncb - 1) * 128 if n_items % 128 else 128

    mesh = plsc.VectorSubcoreMesh(core_axis_name="c", subcore_axis_name="s")

    @functools.partial(
        pl.kernel,
        mesh=mesh,
        out_type=jax.ShapeDtypeStruct((ncb * 32, 128), jnp.float32),
        compiler_params=pltpu.CompilerParams(
            use_tc_tiling_on_sc=True, needs_layout_passes=False),
        scratch_types=[
            pltpu.VMEM((2, 4, 8, KCB * 128), jnp.float32),
            pltpu.VMEM((2, KCB * 32, 128), jnp.float32),
            pltpu.SemaphoreType.DMA,
            pltpu.SemaphoreType.DMA,
            pltpu.SemaphoreType.DMA,
            pltpu.SemaphoreType.DMA,
        ],
    )
    def relayout_kernel(tt_hbm, t2d_hbm, inb, outb, r0, r1, w0, w1):
        w = lax.axis_index("s") * nc + lax.axis_index("c")
        semr = (r0, r1)
        semw = (w0, w1)
        lane = lax.iota(jnp.int32, LANE)
        ebv = [(lane + e0) >> 3 for e0 in (0, 16)]
        erv = [(lane + e0) & 7 for e0 in (0, 16)]

        def issue_read(cb0, k, width=KCB * 128):
            for eb in range(4):
                pltpu.async_copy(
                    tt_hbm.at[pl.ds(eb * 8, 8), pl.ds(cb0 * 128, width)],
                    inb.at[k, eb, :, pl.ds(0, width)], semr[k])

        def wait_read(k, width=KCB * 128):
            for eb in range(4):
                pltpu.make_async_copy(
                    tt_hbm.at[pl.ds(eb * 8, 8), pl.ds(0, width)],
                    inb.at[k, eb, :, pl.ds(0, width)], semr[k]).wait()

        def transpose(k, kcb=KCB):
            for kk in range(kcb):
                for r in range(32):
                    for m in range(8):
                        half = m % 2
                        col = jnp.full(
                            (LANE,), kk * 128 + r * 4 + m // 2, jnp.int32)
                        v = plsc.load_gather(
                            inb.at[k], [ebv[half], erv[half], col])
                        outb[k, kk * 32 + r, pl.ds(16 * m, LANE)] = v

        base = w * ng * KCB
        issue_read(base, 0)
        issue_read(base + KCB, 1)

        def gbody(j, carry):
            for k in range(2):
                g = j * 2 + k
                wait_read(k)

                @pl.when(j > 0)
                def _():
                    pltpu.make_async_copy(
                        outb.at[k], t2d_hbm.at[pl.ds(0, KCB * 32)],
                        semw[k]).wait()

                transpose(k)
                pltpu.async_copy(
                    outb.at[k],
                    t2d_hbm.at[pl.ds((base + g * KCB) * 32, KCB * 32)],
                    semw[k])
                nxt = jnp.minimum(g + 2, ng - 1)
                issue_read(base + nxt * KCB, k)
            return carry

        lax.fori_loop(0, ng // 2, gbody, 0)
        for k in range(2):
            wait_read(k)
            pltpu.make_async_copy(
                outb.at[k], t2d_hbm.at[pl.ds(0, KCB * 32)], semw[k]).wait()

        if n_rest:
            for is_last, width in ((False, 128), (True, last_w)):

                @pl.when((w == n_rest - 1) if is_last else (w < n_rest - 1))
                def _(width=width):
                    cb = ng * KCB * nw + w
                    issue_read(cb, 0, width)
                    wait_read(0, width)
                    transpose(0, 1)
                    pltpu.async_copy(
                        outb.at[0, pl.ds(0, 32)],
                        t2d_hbm.at[pl.ds(cb * 32, 32)], semw[0])
                    pltpu.make_async_copy(
                        outb.at[0, pl.ds(0, 32)],
                        t2d_hbm.at[pl.ds(0, 32)], semw[0]).wait()

    return relayout_kernel


@functools.lru_cache(maxsize=None)
def _make_gather(batch: int, hist: int, n_rows: int):
    info = plsc.get_sparse_core_info()
    nc, ns = info.num_cores, info.num_subcores
    nw = nc * ns
    nbb = batch // BBLK
    assert nbb == nw and batch % BBLK == 0 and hist % RING == 0
    neb = EMB_DIM // 8

    mesh = plsc.VectorSubcoreMesh(core_axis_name="c", subcore_axis_name="s")

    @functools.partial(
        pl.kernel,
        mesh=mesh,
        out_type=jax.ShapeDtypeStruct((hist * neb * nbb * 8, BBLK), jnp.float32),
        compiler_params=pltpu.CompilerParams(
            use_tc_tiling_on_sc=False, needs_layout_passes=False),
        scratch_types=[
            pltpu.VMEM((hist, BBLK), jnp.int32),
            pltpu.VMEM((RING, BBLK, EMB_DIM), jnp.float32),
            pltpu.VMEM((RING, neb, 8, BBLK), jnp.float32),
            [pltpu.SemaphoreType.DMA] * RING,
            [pltpu.SemaphoreType.DMA] * RING,
        ],
    )
    def gather_kernel(idx_hbm, table_hbm, out_hbm, idx_v, rows_v, tile_v,
                      semr, semw):
        w = lax.axis_index("s") * nc + lax.axis_index("c")
        pltpu.sync_copy(idx_hbm.at[:, pl.ds(w * BBLK, BBLK)], idx_v)
        lane = lax.iota(jnp.int32, LANE)

        def issue_gather(h, k):
            pltpu.async_copy(
                table_hbm.at[idx_v.at[h]], rows_v.at[k], semr[k])

        for k in range(RING):
            issue_gather(k, k)

        def body(j, carry):
            for k in range(RING):
                h = j * RING + k
                pltpu.make_async_copy(
                    table_hbm.at[idx_v.at[h]], rows_v.at[k], semr[k]).wait()

                @pl.when(j > 0)
                def _():
                    for eb in range(neb):
                        pltpu.make_async_copy(
                            tile_v.at[k, eb],
                            out_hbm.at[pl.ds(0, 8)], semw[k]).wait()

                for e in range(EMB_DIM):
                    col = jnp.full((LANE,), e, jnp.int32)
                    for i0 in range(BBLK // LANE):
                        v = plsc.load_gather(
                            rows_v.at[k], [lane + i0 * LANE, col])
                        tile_v[k, e // 8, e % 8, pl.ds(i0 * LANE, LANE)] = v
                row0 = ((h * neb) * nbb + w) * 8
                for eb in range(neb):
                    pltpu.async_copy(
                        tile_v.at[k, eb],
                        out_hbm.at[pl.ds(row0 + eb * nbb * 8, 8)], semw[k])
                issue_gather(jnp.minimum(h + RING, hist - 1), k)
            return carry

        lax.fori_loop(0, hist // RING, body, 0)
        for k in range(RING):
            pltpu.make_async_copy(
                table_hbm.at[idx_v.at[0]], rows_v.at[k], semr[k]).wait()
            for eb in range(neb):
                pltpu.make_async_copy(
                    tile_v.at[k, eb], out_hbm.at[pl.ds(0, 8)], semw[k]).wait()

    return gather_kernel


def kernel(indices, table):
    b, h = indices.shape
    n = table.shape[0]
    idx_t = indices.T.astype(jnp.int32)
    t2d = _make_relayout(n)(table.T)
    n_pad = t2d.shape[0] * 4
    tlin = t2d.reshape(n_pad, EMB_DIM)
    out2d = _make_gather(b, h, n_pad)(idx_t, tlin)
    out5 = out2d.reshape(h, EMB_DIM // 8, b // BBLK, 8, BBLK)
    return out5.transpose(2, 4, 0, 1, 3).reshape(b, h, EMB_DIM)


# speedup vs baseline: 1.4677x; 1.3415x over previous
import functools

import jax
import jax.numpy as jnp
from jax import lax
from jax.experimental import pallas as pl
from jax.experimental.pallas import tpu as pltpu
from jax.experimental.pallas import tpu_sc as plsc

EMB_DIM = 32
LANE = 16
BBLK = 128
KCB = 1
RING = 2


@functools.lru_cache(maxsize=None)
def _make_relayout(n_items: int):
    info = plsc.get_sparse_core_info()
    nc, ns = info.num_cores, info.num_subcores
    nw = nc * ns
    ncb = (n_items + 127) // 128
    ng = ncb // (KCB * nw)
    n_rest = ncb - ng * KCB * nw
    last_w = n_items - (ncb - 1) * 128 if n_items % 128 else 128

    mesh = plsc.VectorSubcoreMesh(core_axis_name="c", subcore_axis_name="s")

    @functools.partial(
        pl.kernel,
        mesh=mesh,
        out_type=jax.ShapeDtypeStruct((ncb * 32, 128), jnp.float32),
        compiler_params=pltpu.CompilerParams(
            use_tc_tiling_on_sc=True, needs_layout_passes=False),
        scratch_types=[
            pltpu.VMEM((2, 4, 8, KCB * 128), jnp.float32),
            pltpu.VMEM((2, KCB * 32, 128), jnp.float32),
            pltpu.SemaphoreType.DMA,
            pltpu.SemaphoreType.DMA,
            pltpu.SemaphoreType.DMA,
            pltpu.SemaphoreType.DMA,
        ],
    )
    def relayout_kernel(tt_hbm, t2d_hbm, inb, outb, r0, r1, w0, w1):
        w = lax.axis_index("s") * nc + lax.axis_index("c")
        semr = (r0, r1)
        semw = (w0, w1)
        lane = lax.iota(jnp.int32, LANE)
        ebv = [(lane + e0) >> 3 for e0 in (0, 16)]
        erv = [(lane + e0) & 7 for e0 in (0, 16)]

        def issue_read(cb0, k, width=KCB * 128):
            for eb in range(4):
                pltpu.async_copy(
                    tt_hbm.at[pl.ds(eb * 8, 8), pl.ds(cb0 * 128, width)],
                    inb.at[k, eb, :, pl.ds(0, width)], semr[k])

        def wait_read(k, width=KCB * 128):
            for eb in range(4):
                pltpu.make_async_copy(
                    tt_hbm.at[pl.ds(eb * 8, 8), pl.ds(0, width)],
                    inb.at[k, eb, :, pl.ds(0, width)], semr[k]).wait()

        def transpose(k, kcb=KCB):
            for kk in range(kcb):
                for r0 in range(0, 32, 2):
                    vs = []
                    for r in (r0, r0 + 1):
                        for m in range(8):
                            half = m % 2
                            col = jnp.full(
                                (LANE,), kk * 128 + r * 4 + m // 2, jnp.int32)
                            vs.append(plsc.load_gather(
                                inb.at[k], [ebv[half], erv[half], col]))
                    i = 0
                    for r in (r0, r0 + 1):
                        for m in range(8):
                            outb[k, kk * 32 + r, pl.ds(16 * m, LANE)] = vs[i]
                            i += 1

        base = w * ng * KCB
        issue_read(base, 0)
        issue_read(base + KCB, 1)

        def gbody(j, carry):
            for k in range(2):
                g = j * 2 + k
                wait_read(k)

                @pl.when(j > 0)
                def _():
                    pltpu.make_async_copy(
                        outb.at[k], t2d_hbm.at[pl.ds(0, KCB * 32)],
                        semw[k]).wait()

                transpose(k)
                pltpu.async_copy(
                    outb.at[k],
                    t2d_hbm.at[pl.ds((base + g * KCB) * 32, KCB * 32)],
                    semw[k])
                nxt = jnp.minimum(g + 2, ng - 1)
                issue_read(base + nxt * KCB, k)
            return carry

        lax.fori_loop(0, ng // 2, gbody, 0)
        for k in range(2):
            wait_read(k)
            pltpu.make_async_copy(
                outb.at[k], t2d_hbm.at[pl.ds(0, KCB * 32)], semw[k]).wait()

        if n_rest:
            for is_last, width in ((False, 128), (True, last_w)):

                @pl.when((w == n_rest - 1) if is_last else (w < n_rest - 1))
                def _(width=width):
                    cb = ng * KCB * nw + w
                    issue_read(cb, 0, width)
                    wait_read(0, width)
                    transpose(0, 1)
                    pltpu.async_copy(
                        outb.at[0, pl.ds(0, 32)],
                        t2d_hbm.at[pl.ds(cb * 32, 32)], semw[0])
                    pltpu.make_async_copy(
                        outb.at[0, pl.ds(0, 32)],
                        t2d_hbm.at[pl.ds(0, 32)], semw[0]).wait()

    return relayout_kernel


@functools.lru_cache(maxsize=None)
def _make_gather(batch: int, hist: int, n_rows: int):
    info = plsc.get_sparse_core_info()
    nc, ns = info.num_cores, info.num_subcores
    nw = nc * ns
    nbb = batch // BBLK
    assert nbb == nw and batch % BBLK == 0 and hist % RING == 0
    neb = EMB_DIM // 8

    mesh = plsc.VectorSubcoreMesh(core_axis_name="c", subcore_axis_name="s")

    @functools.partial(
        pl.kernel,
        mesh=mesh,
        out_type=jax.ShapeDtypeStruct((hist * neb * nbb * 8, BBLK), jnp.float32),
        compiler_params=pltpu.CompilerParams(
            use_tc_tiling_on_sc=False, needs_layout_passes=False),
        scratch_types=[
            pltpu.VMEM((hist, BBLK), jnp.int32),
            pltpu.VMEM((RING, BBLK, EMB_DIM), jnp.float32),
            pltpu.VMEM((RING, neb, 8, BBLK), jnp.float32),
            [pltpu.SemaphoreType.DMA] * RING,
            [pltpu.SemaphoreType.DMA] * RING,
        ],
    )
    def gather_kernel(idx_hbm, table_hbm, out_hbm, idx_v, rows_v, tile_v,
                      semr, semw):
        w = lax.axis_index("s") * nc + lax.axis_index("c")
        pltpu.sync_copy(idx_hbm.at[:, pl.ds(w * BBLK, BBLK)], idx_v)
        lane = lax.iota(jnp.int32, LANE)

        def issue_gather(h, k):
            pltpu.async_copy(
                table_hbm.at[idx_v.at[h]], rows_v.at[k], semr[k])

        for k in range(RING):
            issue_gather(k, k)

        def body(j, carry):
            for k in range(RING):
                h = j * RING + k
                pltpu.make_async_copy(
                    table_hbm.at[idx_v.at[h]], rows_v.at[k], semr[k]).wait()

                @pl.when(j > 0)
                def _():
                    for eb in range(neb):
                        pltpu.make_async_copy(
                            tile_v.at[k, eb],
                            out_hbm.at[pl.ds(0, 8)], semw[k]).wait()

                for e0 in range(0, EMB_DIM, 2):
                    vs = []
                    for e in (e0, e0 + 1):
                        col = jnp.full((LANE,), e, jnp.int32)
                        for i0 in range(BBLK // LANE):
                            vs.append(plsc.load_gather(
                                rows_v.at[k], [lane + i0 * LANE, col]))
                    i = 0
                    for e in (e0, e0 + 1):
                        for i0 in range(BBLK // LANE):
                            tile_v[k, e // 8, e % 8,
                                   pl.ds(i0 * LANE, LANE)] = vs[i]
                            i += 1
                row0 = ((h * neb) * nbb + w) * 8
                for eb in range(neb):
                    pltpu.async_copy(
                        tile_v.at[k, eb],
                        out_hbm.at[pl.ds(row0 + eb * nbb * 8, 8)], semw[k])
                issue_gather(jnp.minimum(h + RING, hist - 1), k)
            return carry

        lax.fori_loop(0, hist // RING, body, 0)
        for k in range(RING):
            pltpu.make_async_copy(
                table_hbm.at[idx_v.at[0]], rows_v.at[k], semr[k]).wait()
            for eb in range(neb):
                pltpu.make_async_copy(
                    tile_v.at[k, eb], out_hbm.at[pl.ds(0, 8)], semw[k]).wait()

    return gather_kernel


def kernel(indices, table):
    b, h = indices.shape
    n = table.shape[0]
    idx_t = indices.T.astype(jnp.int32)
    t2d = _make_relayout(n)(table.T)
    n_pad = t2d.shape[0] * 4
    tlin = t2d.reshape(n_pad, EMB_DIM)
    out2d = _make_gather(b, h, n_pad)(idx_t, tlin)
    out5 = out2d.reshape(h, EMB_DIM // 8, b // BBLK, 8, BBLK)
    return out5.transpose(2, 4, 0, 1, 3).reshape(b, h, EMB_DIM)


# speedup vs baseline: 1.5458x; 1.0532x over previous
import functools

import jax
import jax.numpy as jnp
from jax import lax
from jax.experimental import pallas as pl
from jax.experimental.pallas import tpu as pltpu
from jax.experimental.pallas import tpu_sc as plsc

EMB_DIM = 32
LANE = 16
BBLK = 128
KCB = 1
RING = 5


@functools.lru_cache(maxsize=None)
def _make_relayout(n_items: int):
    info = plsc.get_sparse_core_info()
    nc, ns = info.num_cores, info.num_subcores
    nw = nc * ns
    ncb = (n_items + 127) // 128
    ng = ncb // (KCB * nw)
    n_rest = ncb - ng * KCB * nw
    last_w = n_items - (ncb - 1) * 128 if n_items % 128 else 128

    mesh = plsc.VectorSubcoreMesh(core_axis_name="c", subcore_axis_name="s")

    @functools.partial(
        pl.kernel,
        mesh=mesh,
        out_type=jax.ShapeDtypeStruct((ncb * 32, 128), jnp.float32),
        compiler_params=pltpu.CompilerParams(
            use_tc_tiling_on_sc=True, needs_layout_passes=False),
        scratch_types=[
            pltpu.VMEM((4, 4, 8, KCB * 128), jnp.float32),
            pltpu.VMEM((4, KCB * 32, 128), jnp.float32),
            [pltpu.SemaphoreType.DMA] * 4,
            [pltpu.SemaphoreType.DMA] * 4,
        ],
    )
    def relayout_kernel(tt_hbm, t2d_hbm, inb, outb, semr, semw):
        w = lax.axis_index("s") * nc + lax.axis_index("c")
        lane = lax.iota(jnp.int32, LANE)
        ebv = [(lane + e0) >> 3 for e0 in (0, 16)]
        erv = [(lane + e0) & 7 for e0 in (0, 16)]

        def issue_read(cb0, k, width=KCB * 128):
            for eb in range(4):
                pltpu.async_copy(
                    tt_hbm.at[pl.ds(eb * 8, 8), pl.ds(cb0 * 128, width)],
                    inb.at[k, eb, :, pl.ds(0, width)], semr[k])

        def wait_read(k, width=KCB * 128):
            for eb in range(4):
                pltpu.make_async_copy(
                    tt_hbm.at[pl.ds(eb * 8, 8), pl.ds(0, width)],
                    inb.at[k, eb, :, pl.ds(0, width)], semr[k]).wait()

        def transpose(k, kcb=KCB):
            for kk in range(kcb):
                for r0 in range(0, 32, 2):
                    vs = []
                    for r in (r0, r0 + 1):
                        for m in range(8):
                            half = m % 2
                            col = jnp.full(
                                (LANE,), kk * 128 + r * 4 + m // 2, jnp.int32)
                            vs.append(plsc.load_gather(
                                inb.at[k], [ebv[half], erv[half], col]))
                    i = 0
                    for r in (r0, r0 + 1):
                        for m in range(8):
                            outb[k, kk * 32 + r, pl.ds(16 * m, LANE)] = vs[i]
                            i += 1

        base = w * ng * KCB
        for k in range(4):
            issue_read(base + k * KCB, k)

        def gbody(j, carry):
            for k in range(4):
                g = j * 4 + k
                wait_read(k)

                @pl.when(j > 0)
                def _():
                    pltpu.make_async_copy(
                        outb.at[k], t2d_hbm.at[pl.ds(0, KCB * 32)],
                        semw[k]).wait()

                transpose(k)
                pltpu.async_copy(
                    outb.at[k],
                    t2d_hbm.at[pl.ds((base + g * KCB) * 32, KCB * 32)],
                    semw[k])
                nxt = jnp.minimum(g + 4, ng - 1)
                issue_read(base + nxt * KCB, k)
            return carry

        lax.fori_loop(0, ng // 4, gbody, 0)
        for k in range(4):
            wait_read(k)
            pltpu.make_async_copy(
                outb.at[k], t2d_hbm.at[pl.ds(0, KCB * 32)], semw[k]).wait()

        if n_rest:
            for is_last, width in ((False, 128), (True, last_w)):

                @pl.when((w == n_rest - 1) if is_last else (w < n_rest - 1))
                def _(width=width):
                    cb = ng * KCB * nw + w
                    issue_read(cb, 0, width)
                    wait_read(0, width)
                    transpose(0, 1)
                    pltpu.async_copy(
                        outb.at[0, pl.ds(0, 32)],
                        t2d_hbm.at[pl.ds(cb * 32, 32)], semw[0])
                    pltpu.make_async_copy(
                        outb.at[0, pl.ds(0, 32)],
                        t2d_hbm.at[pl.ds(0, 32)], semw[0]).wait()

    return relayout_kernel


@functools.lru_cache(maxsize=None)
def _make_gather(batch: int, hist: int, n_rows: int):
    info = plsc.get_sparse_core_info()
    nc, ns = info.num_cores, info.num_subcores
    nw = nc * ns
    nbb = batch // BBLK
    assert nbb == nw and batch % BBLK == 0 and hist % RING == 0
    neb = EMB_DIM // 8

    mesh = plsc.VectorSubcoreMesh(core_axis_name="c", subcore_axis_name="s")

    @functools.partial(
        pl.kernel,
        mesh=mesh,
        out_type=jax.ShapeDtypeStruct((hist * neb * nbb * 8, BBLK), jnp.float32),
        compiler_params=pltpu.CompilerParams(
            use_tc_tiling_on_sc=False, needs_layout_passes=False),
        scratch_types=[
            pltpu.VMEM((hist, BBLK), jnp.int32),
            pltpu.VMEM((RING, BBLK, EMB_DIM), jnp.float32),
            pltpu.VMEM((RING, neb, 8, BBLK), jnp.float32),
            [pltpu.SemaphoreType.DMA] * RING,
            [pltpu.SemaphoreType.DMA] * RING,
        ],
    )
    def gather_kernel(idx_hbm, table_hbm, out_hbm, idx_v, rows_v, tile_v,
                      semr, semw):
        w = lax.axis_index("s") * nc + lax.axis_index("c")
        pltpu.sync_copy(idx_hbm.at[:, pl.ds(w * BBLK, BBLK)], idx_v)
        lane = lax.iota(jnp.int32, LANE)

        def issue_gather(h, k):
            pltpu.async_copy(
                table_hbm.at[idx_v.at[h]], rows_v.at[k], semr[k])

        for k in range(RING):
            issue_gather(k, k)

        def body(j, carry):
            for k in range(RING):
                h = j * RING + k
                pltpu.make_async_copy(
                    table_hbm.at[idx_v.at[h]], rows_v.at[k], semr[k]).wait()

                @pl.when(j > 0)
                def _():
                    for eb in range(neb):
                        pltpu.make_async_copy(
                            tile_v.at[k, eb],
                            out_hbm.at[pl.ds(0, 8)], semw[k]).wait()

                for e0 in range(0, EMB_DIM, 2):
                    vs = []
                    for e in (e0, e0 + 1):
                        col = jnp.full((LANE,), e, jnp.int32)
                        for i0 in range(BBLK // LANE):
                            vs.append(plsc.load_gather(
                                rows_v.at[k], [lane + i0 * LANE, col]))
                    i = 0
                    for e in (e0, e0 + 1):
                        for i0 in range(BBLK // LANE):
                            tile_v[k, e // 8, e % 8,
                                   pl.ds(i0 * LANE, LANE)] = vs[i]
                            i += 1
                row0 = ((h * neb) * nbb + w) * 8
                for eb in range(neb):
                    pltpu.async_copy(
                        tile_v.at[k, eb],
                        out_hbm.at[pl.ds(row0 + eb * nbb * 8, 8)], semw[k])
                issue_gather(jnp.minimum(h + RING, hist - 1), k)
            return carry

        lax.fori_loop(0, hist // RING, body, 0)
        for k in range(RING):
            pltpu.make_async_copy(
                table_hbm.at[idx_v.at[0]], rows_v.at[k], semr[k]).wait()
            for eb in range(neb):
                pltpu.make_async_copy(
                    tile_v.at[k, eb], out_hbm.at[pl.ds(0, 8)], semw[k]).wait()

    return gather_kernel


def kernel(indices, table):
    b, h = indices.shape
    n = table.shape[0]
    idx_t = indices.T.astype(jnp.int32)
    t2d = _make_relayout(n)(table.T)
    n_pad = t2d.shape[0] * 4
    tlin = t2d.reshape(n_pad, EMB_DIM)
    out2d = _make_gather(b, h, n_pad)(idx_t, tlin)
    out5 = out2d.reshape(h, EMB_DIM // 8, b // BBLK, 8, BBLK)
    return out5.transpose(2, 4, 0, 1, 3).reshape(b, h, EMB_DIM)


# speedup vs baseline: 1.8122x; 1.1723x over previous
import functools

import jax
import jax.numpy as jnp
from jax import lax
from jax.experimental import pallas as pl
from jax.experimental.pallas import tpu as pltpu
from jax.experimental.pallas import tpu_sc as plsc

EMB_DIM = 32
LANE = 16
BBLK = 128
KCB = 1
RING = 2


@functools.lru_cache(maxsize=None)
def _make_relayout(n_items: int):
    info = plsc.get_sparse_core_info()
    nc, ns = info.num_cores, info.num_subcores
    nw = nc * ns
    ncb = (n_items + 127) // 128
    ng = ncb // (KCB * nw)
    n_rest = ncb - ng * KCB * nw
    last_w = n_items - (ncb - 1) * 128 if n_items % 128 else 128

    mesh = plsc.VectorSubcoreMesh(core_axis_name="c", subcore_axis_name="s")

    @functools.partial(
        pl.kernel,
        mesh=mesh,
        out_type=jax.ShapeDtypeStruct((ncb * 32, 128), jnp.float32),
        compiler_params=pltpu.CompilerParams(
            use_tc_tiling_on_sc=True, needs_layout_passes=False),
        scratch_types=[
            pltpu.VMEM((2, 4, 8, 137), jnp.float32),
            pltpu.VMEM((2, KCB * 32, 128), jnp.float32),
            [pltpu.SemaphoreType.DMA] * 2,
            [pltpu.SemaphoreType.DMA] * 2,
        ],
    )
    def relayout_kernel(tt_hbm, t2d_hbm, inb, outb, semr, semw):
        w = lax.axis_index("s") * nc + lax.axis_index("c")
        lane = lax.iota(jnp.int32, LANE)
        ebv = [(lane + e0) >> 3 for e0 in (0, 16)]
        erv = [(lane + e0) & 7 for e0 in (0, 16)]

        def issue_read(cb0, k, width=KCB * 128):
            for eb in range(4):
                pltpu.async_copy(
                    tt_hbm.at[pl.ds(eb * 8, 8), pl.ds(cb0 * 128, width)],
                    inb.at[k, eb, :, pl.ds(0, width)], semr[k])

        def wait_read(k, width=KCB * 128):
            for eb in range(4):
                pltpu.make_async_copy(
                    tt_hbm.at[pl.ds(eb * 8, 8), pl.ds(0, width)],
                    inb.at[k, eb, :, pl.ds(0, width)], semr[k]).wait()

        def transpose(k, kcb=KCB):
            for kk in range(kcb):
                for r0 in range(0, 32, 2):
                    vs = []
                    for r in (r0, r0 + 1):
                        for m in range(8):
                            half = m % 2
                            col = jnp.full(
                                (LANE,), kk * 128 + r * 4 + m // 2, jnp.int32)
                            vs.append(plsc.load_gather(
                                inb.at[k], [ebv[half], erv[half], col]))
                    i = 0
                    for r in (r0, r0 + 1):
                        for m in range(8):
                            outb[k, kk * 32 + r, pl.ds(16 * m, LANE)] = vs[i]
                            i += 1

        base = w * ng * KCB
        for k in range(2):
            issue_read(base + k * KCB, k)

        def gbody(j, carry):
            for k in range(2):
                g = j * 2 + k
                wait_read(k)

                @pl.when(j > 0)
                def _():
                    pltpu.make_async_copy(
                        outb.at[k], t2d_hbm.at[pl.ds(0, KCB * 32)],
                        semw[k]).wait()

                transpose(k)
                pltpu.async_copy(
                    outb.at[k],
                    t2d_hbm.at[pl.ds((base + g * KCB) * 32, KCB * 32)],
                    semw[k])
                nxt = jnp.minimum(g + 2, ng - 1)
                issue_read(base + nxt * KCB, k)
            return carry

        lax.fori_loop(0, ng // 2, gbody, 0)
        for k in range(2):
            wait_read(k)
            pltpu.make_async_copy(
                outb.at[k], t2d_hbm.at[pl.ds(0, KCB * 32)], semw[k]).wait()

        if n_rest:
            cb = ng * KCB * nw + w

            @pl.when(w < n_rest - 1)
            def _():
                issue_read(cb, 0, 128)
                wait_read(0, 128)

            @pl.when(w == n_rest - 1)
            def _():
                issue_read(cb, 0, last_w)
                wait_read(0, last_w)

            @pl.when(w < n_rest)
            def _():
                transpose(0, 1)
                pltpu.async_copy(
                    outb.at[0, pl.ds(0, 32)],
                    t2d_hbm.at[pl.ds(cb * 32, 32)], semw[0])
                pltpu.make_async_copy(
                    outb.at[0, pl.ds(0, 32)],
                    t2d_hbm.at[pl.ds(0, 32)], semw[0]).wait()

    return relayout_kernel


@functools.lru_cache(maxsize=None)
def _make_gather(batch: int, hist: int, n_rows: int):
    info = plsc.get_sparse_core_info()
    nc, ns = info.num_cores, info.num_subcores
    nw = nc * ns
    nbb = batch // BBLK
    assert nbb == nw and batch % BBLK == 0 and hist % RING == 0
    neb = EMB_DIM // 8

    mesh = plsc.VectorSubcoreMesh(core_axis_name="c", subcore_axis_name="s")

    @functools.partial(
        pl.kernel,
        mesh=mesh,
        out_type=jax.ShapeDtypeStruct((hist * neb * nbb * 8, BBLK), jnp.float32),
        compiler_params=pltpu.CompilerParams(
            use_tc_tiling_on_sc=False, needs_layout_passes=False),
        scratch_types=[
            pltpu.VMEM((hist, BBLK), jnp.int32),
            pltpu.VMEM((RING, BBLK, EMB_DIM), jnp.float32),
            pltpu.VMEM((RING, neb, 8, 137), jnp.float32),
            [pltpu.SemaphoreType.DMA] * RING,
            [pltpu.SemaphoreType.DMA] * RING,
        ],
    )
    def gather_kernel(idx_hbm, table_hbm, out_hbm, idx_v, rows_v, tile_v,
                      semr, semw):
        w = lax.axis_index("s") * nc + lax.axis_index("c")
        pltpu.sync_copy(idx_hbm.at[:, pl.ds(w * BBLK, BBLK)], idx_v)
        lane = lax.iota(jnp.int32, LANE)
        ebv = [(lane + e0) >> 3 for e0 in (0, 16)]
        erv = [(lane + e0) & 7 for e0 in (0, 16)]

        def issue_gather(h, k):
            pltpu.async_copy(
                table_hbm.at[idx_v.at[h]], rows_v.at[k], semr[k])

        for k in range(RING):
            issue_gather(k, k)

        def body(j, carry):
            for k in range(RING):
                h = j * RING + k
                pltpu.make_async_copy(
                    table_hbm.at[idx_v.at[h]], rows_v.at[k], semr[k]).wait()

                @pl.when(j > 0)
                def _():
                    for eb in range(neb):
                        pltpu.make_async_copy(
                            tile_v.at[k, eb, :, pl.ds(0, BBLK)],
                            out_hbm.at[pl.ds(0, 8)], semw[k]).wait()

                for it0 in range(0, BBLK, 8):
                    vs = []
                    for it in range(it0, it0 + 8):
                        for half in range(2):
                            vs.append(rows_v[k, it, pl.ds(half * LANE, LANE)])
                    i = 0
                    for it in range(it0, it0 + 8):
                        for half in range(2):
                            ilv = jnp.full((LANE,), it, jnp.int32)
                            plsc.store_scatter(
                                tile_v.at[k], [ebv[half], erv[half], ilv],
                                vs[i])
                            i += 1
                row0 = ((h * neb) * nbb + w) * 8
                for eb in range(neb):
                    pltpu.async_copy(
                        tile_v.at[k, eb, :, pl.ds(0, BBLK)],
                        out_hbm.at[pl.ds(row0 + eb * nbb * 8, 8)], semw[k])
                issue_gather(jnp.minimum(h + RING, hist - 1), k)
            return carry

        lax.fori_loop(0, hist // RING, body, 0)
        for k in range(RING):
            pltpu.make_async_copy(
                table_hbm.at[idx_v.at[0]], rows_v.at[k], semr[k]).wait()
            for eb in range(neb):
                pltpu.make_async_copy(
                    tile_v.at[k, eb, :, pl.ds(0, BBLK)],
                    out_hbm.at[pl.ds(0, 8)], semw[k]).wait()

    return gather_kernel


def kernel(indices, table):
    b, h = indices.shape
    n = table.shape[0]
    idx_t = indices.T.astype(jnp.int32)
    t2d = _make_relayout(n)(table.T)
    n_pad = t2d.shape[0] * 4
    tlin = t2d.reshape(n_pad, EMB_DIM)
    out2d = _make_gather(b, h, n_pad)(idx_t, tlin)
    out5 = out2d.reshape(h, EMB_DIM // 8, b // BBLK, 8, BBLK)
    return out5.transpose(2, 4, 0, 1, 3).reshape(b, h, EMB_DIM)


# speedup vs baseline: 1.9610x; 1.0821x over previous
import functools

import jax
import jax.numpy as jnp
from jax import lax
from jax.experimental import pallas as pl
from jax.experimental.pallas import tpu as pltpu
from jax.experimental.pallas import tpu_sc as plsc

EMB_DIM = 32
LANE = 16
BBLK = 128
KCB = 1
RING = 2


@functools.lru_cache(maxsize=None)
def _make_relayout(n_items: int):
    info = plsc.get_sparse_core_info()
    nc, ns = info.num_cores, info.num_subcores
    nw = nc * ns
    ncb = (n_items + 127) // 128
    ng = ncb // (KCB * nw)
    n_rest = ncb - ng * KCB * nw
    last_w = n_items - (ncb - 1) * 128 if n_items % 128 else 128

    mesh = plsc.VectorSubcoreMesh(core_axis_name="c", subcore_axis_name="s")

    @functools.partial(
        pl.kernel,
        mesh=mesh,
        out_type=jax.ShapeDtypeStruct((ncb * 32, 128), jnp.float32),
        compiler_params=pltpu.CompilerParams(
            use_tc_tiling_on_sc=True, needs_layout_passes=False),
        scratch_types=[
            pltpu.VMEM((2, 4, 8, 128), jnp.float32),
            pltpu.VMEM((2, KCB * 32, 128), jnp.float32),
            [pltpu.SemaphoreType.DMA] * 2,
            [pltpu.SemaphoreType.DMA] * 2,
        ],
    )
    def relayout_kernel(tt_hbm, t2d_hbm, inb, outb, semr, semw):
        w = lax.axis_index("s") * nc + lax.axis_index("c")
        lane = lax.iota(jnp.int32, LANE)
        ebv = [(lane + e0) >> 3 for e0 in (0, 16)]
        erv = [(lane + e0) & 7 for e0 in (0, 16)]

        def issue_read(cb0, k, width=KCB * 128):
            for eb in range(4):
                pltpu.async_copy(
                    tt_hbm.at[pl.ds(eb * 8, 8), pl.ds(cb0 * 128, width)],
                    inb.at[k, eb, :, pl.ds(0, width)], semr[k])

        def wait_read(k, width=KCB * 128):
            for eb in range(4):
                pltpu.make_async_copy(
                    tt_hbm.at[pl.ds(eb * 8, 8), pl.ds(0, width)],
                    inb.at[k, eb, :, pl.ds(0, width)], semr[k]).wait()

        rowq = lane >> 2
        colq = (lane & 3) * 32

        def transpose(k, kcb=KCB):
            for i0 in range(8):
                vs = []
                for eb in range(4):
                    for er in range(8):
                        vs.append(inb[k, eb, er, pl.ds(i0 * LANE, LANE)])
                i = 0
                for eb in range(4):
                    for er in range(8):
                        plsc.store_scatter(
                            outb.at[k], [rowq + i0 * 4, colq + eb * 8 + er],
                            vs[i])
                        i += 1

        base = w * ng * KCB
        for k in range(2):
            issue_read(base + k * KCB, k)

        def gbody(j, carry):
            for k in range(2):
                g = j * 2 + k
                wait_read(k)

                @pl.when(j > 0)
                def _():
                    pltpu.make_async_copy(
                        outb.at[k], t2d_hbm.at[pl.ds(0, KCB * 32)],
                        semw[k]).wait()

                transpose(k)
                pltpu.async_copy(
                    outb.at[k],
                    t2d_hbm.at[pl.ds((base + g * KCB) * 32, KCB * 32)],
                    semw[k])
                nxt = jnp.minimum(g + 2, ng - 1)
                issue_read(base + nxt * KCB, k)
            return carry

        lax.fori_loop(0, ng // 2, gbody, 0)
        for k in range(2):
            wait_read(k)
            pltpu.make_async_copy(
                outb.at[k], t2d_hbm.at[pl.ds(0, KCB * 32)], semw[k]).wait()

        if n_rest:
            cb = ng * KCB * nw + w

            @pl.when(w < n_rest - 1)
            def _():
                issue_read(cb, 0, 128)
                wait_read(0, 128)

            @pl.when(w == n_rest - 1)
            def _():
                issue_read(cb, 0, last_w)
                wait_read(0, last_w)

            @pl.when(w < n_rest)
            def _():
                transpose(0, 1)
                pltpu.async_copy(
                    outb.at[0, pl.ds(0, 32)],
                    t2d_hbm.at[pl.ds(cb * 32, 32)], semw[0])
                pltpu.make_async_copy(
                    outb.at[0, pl.ds(0, 32)],
                    t2d_hbm.at[pl.ds(0, 32)], semw[0]).wait()

    return relayout_kernel


@functools.lru_cache(maxsize=None)
def _make_gather(batch: int, hist: int, n_rows: int):
    info = plsc.get_sparse_core_info()
    nc, ns = info.num_cores, info.num_subcores
    nw = nc * ns
    nbb = batch // BBLK
    assert nbb == nw and batch % BBLK == 0 and hist % RING == 0
    neb = EMB_DIM // 8

    mesh = plsc.VectorSubcoreMesh(core_axis_name="c", subcore_axis_name="s")

    @functools.partial(
        pl.kernel,
        mesh=mesh,
        out_type=jax.ShapeDtypeStruct((hist * neb * nbb * 8, BBLK), jnp.float32),
        compiler_params=pltpu.CompilerParams(
            use_tc_tiling_on_sc=False, needs_layout_passes=False),
        scratch_types=[
            pltpu.VMEM((hist, BBLK), jnp.int32),
            pltpu.VMEM((RING, BBLK, EMB_DIM), jnp.float32),
            pltpu.VMEM((RING, neb, 8, 137), jnp.float32),
            [pltpu.SemaphoreType.DMA] * RING,
            [pltpu.SemaphoreType.DMA] * RING,
        ],
    )
    def gather_kernel(idx_hbm, table_hbm, out_hbm, idx_v, rows_v, tile_v,
                      semr, semw):
        w = lax.axis_index("s") * nc + lax.axis_index("c")
        pltpu.sync_copy(idx_hbm.at[:, pl.ds(w * BBLK, BBLK)], idx_v)
        lane = lax.iota(jnp.int32, LANE)
        ebv = [(lane + e0) >> 3 for e0 in (0, 16)]
        erv = [(lane + e0) & 7 for e0 in (0, 16)]

        def issue_gather(h, k):
            pltpu.async_copy(
                table_hbm.at[idx_v.at[h]], rows_v.at[k], semr[k])

        for k in range(RING):
            issue_gather(k, k)

        def body(j, carry):
            for k in range(RING):
                h = j * RING + k
                pltpu.make_async_copy(
                    table_hbm.at[idx_v.at[h]], rows_v.at[k], semr[k]).wait()

                @pl.when(j > 0)
                def _():
                    for eb in range(neb):
                        pltpu.make_async_copy(
                            tile_v.at[k, eb, :, pl.ds(0, BBLK)],
                            out_hbm.at[pl.ds(0, 8)], semw[k]).wait()

                for it0 in range(0, BBLK, 8):
                    vs = []
                    for it in range(it0, it0 + 8):
                        for half in range(2):
                            vs.append(rows_v[k, it, pl.ds(half * LANE, LANE)])
                    i = 0
                    for it in range(it0, it0 + 8):
                        for half in range(2):
                            ilv = jnp.full((LANE,), it, jnp.int32)
                            plsc.store_scatter(
                                tile_v.at[k], [ebv[half], erv[half], ilv],
                                vs[i])
                            i += 1
                row0 = ((h * neb) * nbb + w) * 8
                for eb in range(neb):
                    pltpu.async_copy(
                        tile_v.at[k, eb, :, pl.ds(0, BBLK)],
                        out_hbm.at[pl.ds(row0 + eb * nbb * 8, 8)], semw[k])
                issue_gather(jnp.minimum(h + RING, hist - 1), k)
            return carry

        lax.fori_loop(0, hist // RING, body, 0)
        for k in range(RING):
            pltpu.make_async_copy(
                table_hbm.at[idx_v.at[0]], rows_v.at[k], semr[k]).wait()
            for eb in range(neb):
                pltpu.make_async_copy(
                    tile_v.at[k, eb, :, pl.ds(0, BBLK)],
                    out_hbm.at[pl.ds(0, 8)], semw[k]).wait()

    return gather_kernel


def kernel(indices, table):
    b, h = indices.shape
    n = table.shape[0]
    idx_t = indices.T.astype(jnp.int32)
    t2d = _make_relayout(n)(table.T)
    n_pad = t2d.shape[0] * 4
    tlin = t2d.reshape(n_pad, EMB_DIM)
    out2d = _make_gather(b, h, n_pad)(idx_t, tlin)
    out5 = out2d.reshape(h, EMB_DIM // 8, b // BBLK, 8, BBLK)
    return out5.transpose(2, 4, 0, 1, 3).reshape(b, h, EMB_DIM)
